# baseline (speedup 1.0000x reference)
import functools

import jax
import jax.numpy as jnp
from jax import lax
from jax.experimental import pallas as pl
from jax.experimental.pallas import tpu as pltpu
from jax.experimental.pallas import tpu_sc as plsc

NC, NS, L = 2, 16, 16
NW = NC * NS

BATCH, FIELDS, D = 4096, 26, 32
VOC = 1000000
FD = FIELDS * D
BRPW = BATCH // NW
NPH = 2
FPH = FIELDS // NPH
SR = FPH * D
KG = BRPW // L


def _rsqrt(v):
    i = lax.bitcast_convert_type(v, jnp.int32)
    y = lax.bitcast_convert_type(jnp.int32(0x5F3759DF) - (i >> 1), jnp.float32)
    for _ in range(3):
        y = y * (1.5 - 0.5 * v * y * y)
    return y


_mesh = plsc.VectorSubcoreMesh(core_axis_name="c", subcore_axis_name="s")


@functools.partial(
    pl.kernel,
    out_type=jax.ShapeDtypeStruct((FD, BATCH), jnp.float32),
    mesh=_mesh,
    scratch_types=[
        pltpu.VMEM((FIELDS, BRPW), jnp.int32),
        pltpu.VMEM((2, D, BRPW), jnp.int32),
        pltpu.VMEM((2, D, BRPW), jnp.float32),
        pltpu.VMEM((SR, BRPW), jnp.float32),
        pltpu.VMEM((D,), jnp.float32),
        pltpu.VMEM((D,), jnp.float32),
        pltpu.SemaphoreType.DMA,
    ],
)
def _embed_ln(xt_hbm, table_hbm, gamma_hbm, beta_hbm, out_hbm,
              idx_v, eidx_v, g_v, slab_v, gamma_v, beta_v, gsem):
    wid = lax.axis_index("s") * NC + lax.axis_index("c")
    col0 = wid * BRPW

    pltpu.sync_copy(xt_hbm.at[:, pl.ds(col0, BRPW)], idx_v)
    pltpu.sync_copy(gamma_hbm, gamma_v)
    pltpu.sync_copy(beta_hbm, beta_v)

    g_lo = gamma_v[pl.ds(0, L)]
    g_hi = gamma_v[pl.ds(L, L)]
    b_lo = beta_v[pl.ds(0, L)]
    b_hi = beta_v[pl.ds(L, L)]

    def build_and_fire(f, par):
        for k in range(KG):
            iv = idx_v[f, pl.ds(k * L, L)]
            for d in range(D):
                eidx_v[par, d, pl.ds(k * L, L)] = iv + d * VOC
        for d in range(D):
            pltpu.async_copy(
                table_hbm.at[eidx_v.at[par, d]], g_v.at[par, d], gsem)

    def wait_field(par):
        for d in range(D):
            pltpu.make_async_copy(
                table_hbm.at[eidx_v.at[par, d]], g_v.at[par, d], gsem).wait()

    build_and_fire(0, 0)

    def field_body(f, _):
        par = f & 1
        fl = jnp.where(f >= FPH, f - FPH, f)

        @pl.when(f + 1 < FIELDS)
        def _():
            build_and_fire(f + 1, par ^ 1)

        wait_field(par)

        for k in range(KG):
            s = jnp.zeros((L,), jnp.float32)
            ss = jnp.zeros((L,), jnp.float32)
            for d in range(D):
                g = g_v[par, d, pl.ds(k * L, L)]
                s = s + g
                ss = ss + g * g
            mean = s * (1.0 / D)
            var = ss * (1.0 / D) - mean * mean
            rstd = _rsqrt(var + 1e-5)
            for d in range(D):
                g = g_v[par, d, pl.ds(k * L, L)]
                gam = g_lo[d] if d < L else g_hi[d - L]
                bet = b_lo[d] if d < L else b_hi[d - L]
                slab_v[fl * D + d, pl.ds(k * L, L)] = (g - mean) * rstd * gam + bet

        @pl.when(jnp.logical_or(f == FPH - 1, f == FIELDS - 1))
        def _():
            row0 = pl.multiple_of(jnp.where(f >= FPH, SR, 0), 8)
            pltpu.sync_copy(
                slab_v, out_hbm.at[pl.ds(row0, SR), pl.ds(col0, BRPW)])
        return 0

    lax.fori_loop(0, FIELDS, field_body, 0)


def kernel(x, table, gamma, beta):
    xt = x.T
    t1d = table.T.reshape(VOC * D)
    out_t = _embed_ln(xt, t1d, gamma, beta)
    return out_t.T.reshape(BATCH, FD)
